# baseline (speedup 1.0000x reference)
import functools

import jax
import jax.numpy as jnp
from jax import lax
from jax.experimental import pallas as pl
from jax.experimental.pallas import tpu as pltpu
from jax.experimental.pallas import tpu_sc as plsc

_B, _L, _D = 4096, 128, 64
_NC, _NS = 2, 16
_NW = _NC * _NS
_BPW = _B // _NW
_NBUF = 2


@functools.partial(
    pl.kernel,
    out_type=jax.ShapeDtypeStruct((_B * _L, _D), jnp.float32),
    mesh=plsc.VectorSubcoreMesh(
        core_axis_name="c", subcore_axis_name="s",
        num_cores=_NC, num_subcores=_NS,
    ),
    scratch_types=[
        pltpu.VMEM((_L,), jnp.int32),
        pltpu.VMEM((_NBUF, _L), jnp.int32),
        pltpu.VMEM((_NBUF, _L, _D), jnp.float32),
        pltpu.SemaphoreType.DMA,
        pltpu.SemaphoreType.DMA,
    ],
    compiler_params=pltpu.CompilerParams(use_tc_tiling_on_sc=False),
)
def _sc_interleave(x_hbm, p_hbm, out_hbm, p_v, idx_v, rows_v, sem0, sem1):
    sems = (sem0, sem1)
    wid = lax.axis_index("s") * _NC + lax.axis_index("c")
    b0 = wid * _BPW

    pltpu.sync_copy(p_hbm, p_v)

    def fill_idx(slot, b):
        base = (b0 + b) * _L
        for j in range(_L // 16):
            sl = pl.ds(j * 16, 16)
            idx_v[slot, sl] = p_v[sl] + base

    def gather(slot):
        return pltpu.make_async_copy(
            x_hbm.at[idx_v.at[slot]], rows_v.at[slot], sems[slot])

    def store(slot, b):
        pltpu.sync_copy(rows_v.at[slot], out_hbm.at[pl.ds((b0 + b) * _L, _L)])

    ngroups = _BPW // _NBUF

    fill_idx(0, 0)
    gather(0).start()

    def group(g, carry):
        b_even = g * _NBUF
        fill_idx(1, b_even + 1)
        gather(1).start()
        gather(0).wait()
        store(0, b_even)

        @pl.when(g + 1 < ngroups)
        def _():
            fill_idx(0, b_even + _NBUF)
            gather(0).start()

        gather(1).wait()
        store(1, b_even + 1)
        return carry

    lax.fori_loop(0, ngroups, group, 0)


def kernel(inputs, p_array):
    x_flat = inputs.reshape(_B * _L, _D)
    out = _sc_interleave(x_flat, p_array)
    return out.reshape(_B, _L, _D)


# speedup vs baseline: 1.0212x; 1.0212x over previous
import functools

import jax
import jax.numpy as jnp
from jax import lax
from jax.experimental import pallas as pl
from jax.experimental.pallas import tpu as pltpu
from jax.experimental.pallas import tpu_sc as plsc

_B, _L, _D = 4096, 128, 64
_NC, _NS = 2, 16
_NW = _NC * _NS
_BPW = _B // _NW
_NBUF = 4


@functools.partial(
    pl.kernel,
    out_type=jax.ShapeDtypeStruct((_B * _L, _D), jnp.float32),
    mesh=plsc.VectorSubcoreMesh(
        core_axis_name="c", subcore_axis_name="s",
        num_cores=_NC, num_subcores=_NS,
    ),
    scratch_types=[
        pltpu.VMEM((_L,), jnp.int32),
        pltpu.VMEM((_NBUF, _L), jnp.int32),
        pltpu.VMEM((_NBUF, _L, _D), jnp.float32),
        [pltpu.SemaphoreType.DMA] * _NBUF,
        [pltpu.SemaphoreType.DMA] * _NBUF,
    ],
    compiler_params=pltpu.CompilerParams(use_tc_tiling_on_sc=False),
)
def _sc_interleave(x_hbm, p_hbm, out_hbm, p_v, idx_v, rows_v, gsems, ssems):
    wid = lax.axis_index("s") * _NC + lax.axis_index("c")
    b0 = wid * _BPW

    pltpu.sync_copy(p_hbm, p_v)

    def fill_idx(slot, b):
        base = (b0 + b) * _L
        for j in range(_L // 16):
            sl = pl.ds(j * 16, 16)
            idx_v[slot, sl] = p_v[sl] + base

    def gather(slot):
        return pltpu.make_async_copy(
            x_hbm.at[idx_v.at[slot]], rows_v.at[slot], gsems[slot])

    def store(slot, b):
        return pltpu.make_async_copy(
            rows_v.at[slot], out_hbm.at[pl.ds((b0 + b) * _L, _L)],
            ssems[slot])

    ngroups = _BPW // _NBUF

    def group(g, carry):
        base = g * _NBUF
        for j in range(_NBUF):
            @pl.when(g > 0)
            def _():
                store(j, base + j).wait()
            fill_idx(j, base + j)
            gather(j).start()
        for j in range(_NBUF):
            gather(j).wait()
            store(j, base + j).start()
        return carry

    lax.fori_loop(0, ngroups, group, 0)
    last = (ngroups - 1) * _NBUF
    for j in range(_NBUF):
        store(j, last + j).wait()


def kernel(inputs, p_array):
    x_flat = inputs.reshape(_B * _L, _D)
    out = _sc_interleave(x_flat, p_array)
    return out.reshape(_B, _L, _D)
